# baseline (speedup 1.0000x reference)
import functools

import jax
import jax.numpy as jnp
from jax import lax
from jax.experimental import pallas as pl
from jax.experimental.pallas import tpu as pltpu
from jax.experimental.pallas import tpu_sc as plsc

S = 384
CZ = 256
VBINS = 2 * (S - 1) + 1
VPAD = 768
APAD = 776


def _emb8_body(w_ref, b_ref, out_ref):
    t = w_ref[...].T + b_ref[...]
    for k in range(8):
        out_ref[k, pl.ds(k, VPAD), :] = t


def _build_emb8(w_pad, b2):
    return pl.pallas_call(
        _emb8_body,
        out_shape=jax.ShapeDtypeStruct((8, APAD, CZ), jnp.float32),
    )(w_pad, b2)


def _make_sc_writer():
    info = plsc.get_sparse_core_info()
    nc, ns = info.num_cores, info.num_subcores
    nw = nc * ns
    rows_per_w = S // nw
    mesh = plsc.VectorSubcoreMesh(core_axis_name="c", subcore_axis_name="s")

    @functools.partial(
        pl.kernel,
        mesh=mesh,
        out_type=jax.ShapeDtypeStruct((S, S, CZ), jnp.float32),
        scratch_types=[
            pltpu.VMEM_SHARED((8, APAD, CZ), jnp.float32),
            pltpu.SemaphoreType.DMA,
            pltpu.SemaphoreType.DMA,
        ],
    )
    def sc_writer(emb8_hbm, out_hbm, spmem, semd, semh):
        cid = lax.axis_index("c")
        sid = lax.axis_index("s")
        half = sid % 2
        kcopy = sid // 2
        r0 = pl.multiple_of(half * 384, 8)
        pltpu.sync_copy(
            emb8_hbm.at[kcopy, pl.ds(r0, 392), :],
            spmem.at[kcopy, pl.ds(r0, 392), :],
        )
        plsc.subcore_barrier()
        wid = sid * nc + cid

        def src_off(r):
            i = wid * rows_per_w + r
            v = (S - 1) - i
            k = (8 - lax.rem(v, 8)) % 8
            off = pl.multiple_of(v + k, 8)
            return i, k, off

        copies = []
        for r in range(rows_per_w):
            i, k, off = src_off(r)
            if r % 2 == 0:
                copies.append(
                    pltpu.async_copy(
                        spmem.at[k, pl.ds(off, S), :], out_hbm.at[i], semd
                    )
                )
            else:
                copies.append(
                    pltpu.async_copy(
                        emb8_hbm.at[k, pl.ds(off, S), :], out_hbm.at[i], semh
                    )
                )
        for cp in copies:
            cp.wait()

    return sc_writer


_SC_WRITER = None


def _get_sc_writer():
    global _SC_WRITER
    if _SC_WRITER is None:
        _SC_WRITER = _make_sc_writer()
    return _SC_WRITER


def kernel(seq_len, ResInd, Wp_w, Wp_b):
    sc_writer = _get_sc_writer()
    w_pad = jnp.pad(Wp_w, ((0, 0), (0, VPAD - VBINS)))
    emb8 = _build_emb8(w_pad, Wp_b.reshape(1, CZ))
    return sc_writer(emb8)


# speedup vs baseline: 21.6512x; 21.6512x over previous
import functools

import jax
import jax.numpy as jnp
from jax import lax
from jax.experimental import pallas as pl
from jax.experimental.pallas import tpu as pltpu
from jax.experimental.pallas import tpu_sc as plsc

S = 384
CZ = 256
VBINS = 2 * (S - 1) + 1
VPAD = 768
APAD = 776


def _emb8_body(w_ref, b_ref, out_ref):
    t = w_ref[...].T + b_ref[...]
    for k in range(8):
        out_ref[k, pl.ds(k, VPAD), :] = t


def _build_emb8(w_pad, b2):
    return pl.pallas_call(
        _emb8_body,
        out_shape=jax.ShapeDtypeStruct((8, APAD, CZ), jnp.float32),
    )(w_pad, b2)


def _make_sc_writer():
    info = plsc.get_sparse_core_info()
    nc, ns = info.num_cores, info.num_subcores
    nw = nc * ns
    rows_per_w = S // nw
    mesh = plsc.VectorSubcoreMesh(core_axis_name="c", subcore_axis_name="s")

    @functools.partial(
        pl.kernel,
        mesh=mesh,
        out_type=jax.ShapeDtypeStruct((S, S, CZ), jnp.float32),
        scratch_types=[
            pltpu.VMEM_SHARED((8, APAD, CZ), jnp.float32),
            pltpu.SemaphoreType.DMA,
        ],
    )
    def sc_writer(emb8_hbm, out_hbm, spmem, semd):
        cid = lax.axis_index("c")
        sid = lax.axis_index("s")
        half = sid % 2
        kcopy = sid // 2
        r0 = pl.multiple_of(half * 384, 8)
        pltpu.sync_copy(
            emb8_hbm.at[kcopy, pl.ds(r0, 392), :],
            spmem.at[kcopy, pl.ds(r0, 392), :],
        )
        plsc.subcore_barrier()
        wid = sid * nc + cid

        def src_off(r):
            i = wid * rows_per_w + r
            v = (S - 1) - i
            k = (8 - lax.rem(v, 8)) % 8
            off = pl.multiple_of(v + k, 8)
            return i, k, off

        copies = []
        for r in range(rows_per_w):
            i, k, off = src_off(r)
            copies.append(
                pltpu.async_copy(
                    spmem.at[k, pl.ds(off, S), :], out_hbm.at[i], semd
                )
            )
        for cp in copies:
            cp.wait()

    return sc_writer


_SC_WRITER = None


def _get_sc_writer():
    global _SC_WRITER
    if _SC_WRITER is None:
        _SC_WRITER = _make_sc_writer()
    return _SC_WRITER


def kernel(seq_len, ResInd, Wp_w, Wp_b):
    sc_writer = _get_sc_writer()
    w_pad = jnp.pad(Wp_w, ((0, 0), (0, VPAD - VBINS)))
    emb8 = _build_emb8(w_pad, Wp_b.reshape(1, CZ))
    return sc_writer(emb8)


# speedup vs baseline: 27.1157x; 1.2524x over previous
import functools

import jax
import jax.numpy as jnp
from jax import lax
from jax.experimental import pallas as pl
from jax.experimental.pallas import tpu as pltpu
from jax.experimental.pallas import tpu_sc as plsc

S = 384
CZ = 256
VBINS = 2 * (S - 1) + 1
VPAD = 768
APAD = 776
NTC = 288
RPS = 8
SCROWS = S - NTC
VMAX = 488


def _emb8_body(w_ref, b_ref, out_ref):
    t = w_ref[...].T + b_ref[...]
    for k in range(8):
        out_ref[k, pl.ds(k, VPAD), :] = t


def _build_emb8(w_pad, b2):
    return pl.pallas_call(
        _emb8_body,
        out_shape=jax.ShapeDtypeStruct((8, APAD, CZ), jnp.float32),
    )(w_pad, b2)


def _tc_body(emb_ref, out_ref):
    i0 = pl.program_id(0) * RPS
    for r in range(RPS):
        v = (S - 1) - (i0 + r)
        k = (8 - v % 8) % 8
        off = pl.multiple_of(v + k, 8)
        out_ref[r] = emb_ref[k, pl.ds(off, S), :]


def _tc_copy(emb_all):
    return pl.pallas_call(
        _tc_body,
        grid=(NTC // RPS,),
        in_specs=[pl.BlockSpec((8, APAD, CZ), lambda i: (0, 0, 0))],
        out_specs=pl.BlockSpec((RPS, S, CZ), lambda i: (i, 0, 0)),
        out_shape=jax.ShapeDtypeStruct((S, S, CZ), jnp.float32),
    )(emb_all)


def _make_sc_writer():
    info = plsc.get_sparse_core_info()
    nc, ns = info.num_cores, info.num_subcores
    nw = nc * ns
    rows_per_w = SCROWS // nw
    mesh = plsc.VectorSubcoreMesh(core_axis_name="c", subcore_axis_name="s")

    @functools.partial(
        pl.kernel,
        mesh=mesh,
        out_type=(),
        scratch_types=[
            pltpu.VMEM_SHARED((8, VMAX, CZ), jnp.float32),
            pltpu.SemaphoreType.DMA,
        ],
    )
    def sc_writer(emb8_hbm, out_hbm, spmem, semd):
        cid = lax.axis_index("c")
        sid = lax.axis_index("s")
        half = sid % 2
        kcopy = sid // 2
        r0 = pl.multiple_of(half * 240, 8)
        pltpu.sync_copy(
            emb8_hbm.at[kcopy, pl.ds(r0, 248), :],
            spmem.at[kcopy, pl.ds(r0, 248), :],
        )
        plsc.subcore_barrier()
        wid = sid * nc + cid
        copies = []
        for r in range(rows_per_w):
            i = NTC + wid * rows_per_w + r
            v = (S - 1) - i
            k = (8 - lax.rem(v, 8)) % 8
            off = pl.multiple_of(v + k, 8)
            copies.append(
                pltpu.async_copy(
                    spmem.at[k, pl.ds(off, S), :], out_hbm.at[i], semd
                )
            )
        for cp in copies:
            cp.wait()

    return sc_writer


_SC_WRITER = None


def _get_sc_writer():
    global _SC_WRITER
    if _SC_WRITER is None:
        _SC_WRITER = _make_sc_writer()
    return _SC_WRITER


def kernel(seq_len, ResInd, Wp_w, Wp_b):
    sc_writer = _get_sc_writer()
    w_pad = jnp.pad(Wp_w, ((0, 0), (0, VPAD - VBINS)))
    emb8 = _build_emb8(w_pad, Wp_b.reshape(1, CZ))
    tc_out = _tc_copy(emb8)
    out_ref = jax.new_ref(tc_out)
    sc_writer(emb8, out_ref)
    return out_ref[...]


# speedup vs baseline: 28.9478x; 1.0676x over previous
import functools

import jax
import jax.numpy as jnp
from jax import lax
from jax.experimental import pallas as pl
from jax.experimental.pallas import tpu as pltpu
from jax.experimental.pallas import tpu_sc as plsc

S = 384
CZ = 256
VBINS = 2 * (S - 1) + 1
VPAD = 768
APAD = 776
NTC = 352
RPS = 8
SCROWS = S - NTC
VMAX = 424


def _emb8_body(w_ref, b_ref, out_ref):
    t = w_ref[...].T + b_ref[...]
    for k in range(8):
        out_ref[k, pl.ds(k, VPAD), :] = t


def _build_emb8(w_pad, b2):
    return pl.pallas_call(
        _emb8_body,
        out_shape=jax.ShapeDtypeStruct((8, APAD, CZ), jnp.float32),
    )(w_pad, b2)


def _tc_body(emb_ref, out_ref):
    i0 = pl.program_id(0) * RPS
    for r in range(RPS):
        v = (S - 1) - (i0 + r)
        k = (8 - v % 8) % 8
        off = pl.multiple_of(v + k, 8)
        out_ref[r] = emb_ref[k, pl.ds(off, S), :]


def _tc_copy(emb_all):
    return pl.pallas_call(
        _tc_body,
        grid=(NTC // RPS,),
        in_specs=[pl.BlockSpec((8, APAD, CZ), lambda i: (0, 0, 0))],
        out_specs=pl.BlockSpec((RPS, S, CZ), lambda i: (i, 0, 0)),
        out_shape=jax.ShapeDtypeStruct((S, S, CZ), jnp.float32),
    )(emb_all)


def _make_sc_writer():
    info = plsc.get_sparse_core_info()
    nc, ns = info.num_cores, info.num_subcores
    nw = nc * ns
    rows_per_w = SCROWS // nw
    mesh = plsc.VectorSubcoreMesh(core_axis_name="c", subcore_axis_name="s")

    @functools.partial(
        pl.kernel,
        mesh=mesh,
        out_type=(),
        scratch_types=[
            pltpu.VMEM_SHARED((8, VMAX, CZ), jnp.float32),
            pltpu.SemaphoreType.DMA,
        ],
    )
    def sc_writer(emb8_hbm, out_hbm, spmem, semd):
        cid = lax.axis_index("c")
        sid = lax.axis_index("s")
        half = sid % 2
        kcopy = sid // 2
        r0 = pl.multiple_of(half * 208, 8)
        pltpu.sync_copy(
            emb8_hbm.at[kcopy, pl.ds(r0, 216), :],
            spmem.at[kcopy, pl.ds(r0, 216), :],
        )
        plsc.subcore_barrier()
        wid = sid * nc + cid
        copies = []
        for r in range(rows_per_w):
            i = NTC + wid * rows_per_w + r
            v = (S - 1) - i
            k = (8 - lax.rem(v, 8)) % 8
            off = pl.multiple_of(v + k, 8)
            copies.append(
                pltpu.async_copy(
                    spmem.at[k, pl.ds(off, S), :], out_hbm.at[i], semd
                )
            )
        for cp in copies:
            cp.wait()

    return sc_writer


_SC_WRITER = None


def _get_sc_writer():
    global _SC_WRITER
    if _SC_WRITER is None:
        _SC_WRITER = _make_sc_writer()
    return _SC_WRITER


def kernel(seq_len, ResInd, Wp_w, Wp_b):
    sc_writer = _get_sc_writer()
    w_pad = jnp.pad(Wp_w, ((0, 0), (0, VPAD - VBINS)))
    emb8 = _build_emb8(w_pad, Wp_b.reshape(1, CZ))
    tc_out = _tc_copy(emb8)
    out_ref = jax.new_ref(tc_out)
    sc_writer(emb8, out_ref)
    return out_ref[...]


# speedup vs baseline: 29.5290x; 1.0201x over previous
import functools

import jax
import jax.numpy as jnp
from jax import lax
from jax.experimental import pallas as pl
from jax.experimental.pallas import tpu as pltpu
from jax.experimental.pallas import tpu_sc as plsc

S = 384
CZ = 256
VBINS = 2 * (S - 1) + 1
VPAD = 768
APAD = 776
NTC = 368
RPS = 8
SCROWS = S - NTC
VMAX = 408


def _emb8_body(w_ref, b_ref, out_ref):
    t = w_ref[...].T + b_ref[...]
    for k in range(8):
        out_ref[k, pl.ds(k, VPAD), :] = t


def _build_emb8(w_pad, b2):
    return pl.pallas_call(
        _emb8_body,
        out_shape=jax.ShapeDtypeStruct((8, APAD, CZ), jnp.float32),
    )(w_pad, b2)


def _tc_body(emb_ref, out_ref):
    i0 = pl.program_id(0) * RPS
    for r in range(RPS):
        v = (S - 1) - (i0 + r)
        k = (8 - v % 8) % 8
        off = pl.multiple_of(v + k, 8)
        out_ref[r] = emb_ref[k, pl.ds(off, S), :]


def _tc_copy(emb_all):
    return pl.pallas_call(
        _tc_body,
        grid=(NTC // RPS,),
        in_specs=[pl.BlockSpec((8, APAD, CZ), lambda i: (0, 0, 0))],
        out_specs=pl.BlockSpec((RPS, S, CZ), lambda i: (i, 0, 0)),
        out_shape=jax.ShapeDtypeStruct((S, S, CZ), jnp.float32),
    )(emb_all)


def _make_sc_writer():
    info = plsc.get_sparse_core_info()
    nc, ns = info.num_cores, info.num_subcores
    nw = nc * ns
    rows_per_w = SCROWS // nw
    mesh = plsc.VectorSubcoreMesh(core_axis_name="c", subcore_axis_name="s")

    @functools.partial(
        pl.kernel,
        mesh=mesh,
        out_type=(),
        scratch_types=[
            pltpu.VMEM_SHARED((8, VMAX, CZ), jnp.float32),
            pltpu.SemaphoreType.DMA,
        ],
    )
    def sc_writer(emb8_hbm, out_hbm, spmem, semd):
        cid = lax.axis_index("c")
        sid = lax.axis_index("s")
        half = sid % 2
        kcopy = sid // 2
        r0 = pl.multiple_of(half * 200, 8)
        pltpu.sync_copy(
            emb8_hbm.at[kcopy, pl.ds(r0, 208), :],
            spmem.at[kcopy, pl.ds(r0, 208), :],
        )
        plsc.subcore_barrier()
        wid = sid * nc + cid

        @pl.when(wid < SCROWS)
        def _write_row():
            i = NTC + wid
            v = (S - 1) - i
            k = (8 - lax.rem(v, 8)) % 8
            off = pl.multiple_of(v + k, 8)
            pltpu.async_copy(
                spmem.at[k, pl.ds(off, S), :], out_hbm.at[i], semd
            ).wait()

    return sc_writer


_SC_WRITER = None


def _get_sc_writer():
    global _SC_WRITER
    if _SC_WRITER is None:
        _SC_WRITER = _make_sc_writer()
    return _SC_WRITER


def kernel(seq_len, ResInd, Wp_w, Wp_b):
    sc_writer = _get_sc_writer()
    w_pad = jnp.pad(Wp_w, ((0, 0), (0, VPAD - VBINS)))
    emb8 = _build_emb8(w_pad, Wp_b.reshape(1, CZ))
    tc_out = _tc_copy(emb8)
    out_ref = jax.new_ref(tc_out)
    sc_writer(emb8, out_ref)
    return out_ref[...]
